# baseline (speedup 1.0000x reference)
import functools

import jax
import jax.numpy as jnp
from jax import lax
from jax.experimental import pallas as pl
from jax.experimental.pallas import tpu as pltpu
from jax.experimental.pallas import tpu_sc as plsc

N_X = 100000
N_Y = 10000
NYP = 10240
E = 320000
D = 128
W = 144
VOCAB = 1000
VP = 1024
B = 64
NC, NS = 2, 16
NTILES = NC * NS
EPT = E // NTILES
CH = 400
NPT = NYP // NTILES
NPS = NYP // NS

_MESH = plsc.VectorSubcoreMesh(
    core_axis_name="c", subcore_axis_name="s", num_cores=NC, num_subcores=NS)
_SC_PARAMS = pltpu.CompilerParams(needs_layout_passes=False,
                                  use_tc_tiling_on_sc=False)


def _wid():
    return lax.axis_index("s") * NC + lax.axis_index("c")


@functools.partial(
    pl.kernel,
    out_type=jax.ShapeDtypeStruct((NYP, D), jnp.float32),
    mesh=_MESH,
    compiler_params=_SC_PARAMS,
    scratch_types=[pltpu.VMEM((NPT,), jnp.int32),
                   pltpu.VMEM((NPT, D), jnp.float32)],
)
def _sc_gather_rows(emb_hbm, idx_hbm, out_hbm, idx_v, rows_v):
    base = _wid() * NPT
    pltpu.sync_copy(idx_hbm.at[pl.ds(base, NPT)], idx_v)
    pltpu.sync_copy(emb_hbm.at[idx_v], rows_v)
    pltpu.sync_copy(rows_v, out_hbm.at[pl.ds(base, NPT)])


CHE = 2000


@functools.partial(
    pl.kernel,
    out_type=(jax.ShapeDtypeStruct((E,), jnp.float32),
              jax.ShapeDtypeStruct((NC, NYP, 8), jnp.float32)),
    mesh=_MESH,
    compiler_params=_SC_PARAMS,
    scratch_types=[
        pltpu.VMEM((NYP,), jnp.float32),
        pltpu.VMEM((NYP,), jnp.float32),
        pltpu.VMEM((16,), jnp.float32),
        pltpu.VMEM((16,), jnp.float32),
        pltpu.VMEM((CHE,), jnp.int32),
        pltpu.VMEM((CHE,), jnp.int32),
        pltpu.VMEM((CHE,), jnp.int32),
        pltpu.VMEM((CHE,), jnp.float32),
        pltpu.VMEM((CHE, 8), jnp.float32),
        pltpu.SemaphoreType.DMA,
        pltpu.SemaphoreType.DMA,
        pltpu.SemaphoreType.DMA,
        pltpu.VMEM_SHARED((N_Y, 8), jnp.float32),
    ],
)
def _sc_escore(s1_hbm, s2_hbm, t_hbm, sh_hbm, src_hbm, dst_hbm, et_hbm,
               ex_out, den8_out, s1_v, s2_v, t_v, sh_v, src_v, dst_v, et_v,
               ex_v, d8_v, isem1, isem2, isem3, d8_sp):
    cid = lax.axis_index("c")
    sid = lax.axis_index("s")
    wid = sid * NC + cid
    ebase = wid * EPT
    pltpu.sync_copy(s1_hbm, s1_v)
    pltpu.sync_copy(s2_hbm, s2_v)
    pltpu.sync_copy(t_hbm, t_v)
    pltpu.sync_copy(sh_hbm, sh_v)
    shv = sh_v[...]
    lane = jax.lax.iota(jnp.int32, 16)
    zidx = jnp.zeros((16,), jnp.int32)
    zval = jnp.zeros((16,), jnp.float32)
    zrow = lane // 8
    zcol = lane & 7

    def _zero2(g, _):
        plsc.store_scatter(d8_v, [zrow + g * 2, zcol], zval)
        return 0

    lax.fori_loop(0, CHE // 2, _zero2, 0)
    sbase = sid * NSPS
    pltpu.sync_copy(d8_v.at[pl.ds(0, NSPS)], d8_sp.at[pl.ds(sbase, NSPS)])
    plsc.subcore_barrier()

    def _chunk(ci, _):
        cbase = ci * CHE
        c1 = pltpu.async_copy(src_hbm.at[pl.ds(ebase + cbase, CHE)], src_v,
                              isem1)
        c2 = pltpu.async_copy(dst_hbm.at[pl.ds(ebase + cbase, CHE)], dst_v,
                              isem2)
        c3 = pltpu.async_copy(et_hbm.at[pl.ds(ebase + cbase, CHE)], et_v,
                              isem3)
        c1.wait()
        c2.wait()
        c3.wait()

        @plsc.parallel_loop(0, CHE // 16, unroll=4)
        def _group(g):
            i0 = g * 16
            i16 = lane + i0
            et16 = et_v[pl.ds(i0, 16)]
            e = (plsc.load_gather(s1_v, [src_v[pl.ds(i0, 16)]])
                 + plsc.load_gather(s2_v, [dst_v[pl.ds(i0, 16)]])
                 + plsc.load_gather(t_v, [et16]))
            e = jnp.where(e >= 0.0, e, e * 0.2)
            ex = jnp.exp(e - shv)
            ex_v[pl.ds(i0, 16)] = ex
            plsc.store_scatter(d8_v, [i16, zidx], ex)
            plsc.store_scatter(d8_v, [i16, et16 + 1], ex)

        pltpu.sync_copy(ex_v, ex_out.at[pl.ds(ebase + cbase, CHE)])
        pltpu.sync_copy(d8_v, d8_sp.at[dst_v], add=True)

        @plsc.parallel_loop(0, CHE // 16, unroll=4)
        def _unwrite(g):
            i0 = g * 16
            i16 = lane + i0
            et16 = et_v[pl.ds(i0, 16)]
            plsc.store_scatter(d8_v, [i16, zidx], zval)
            plsc.store_scatter(d8_v, [i16, et16 + 1], zval)

        return 0

    lax.fori_loop(0, EPT // CHE, _chunk, 0)
    plsc.subcore_barrier()
    pltpu.sync_copy(d8_sp.at[pl.ds(sbase, NSPS)],
                    den8_out.at[cid, pl.ds(sbase, NSPS)])


CHB = 80
NCHB = EPT // CHB
NSPS = N_Y // NS


@functools.partial(
    pl.kernel,
    out_type=jax.ShapeDtypeStruct((NC, NYP, D), jnp.float32),
    mesh=_MESH,
    compiler_params=_SC_PARAMS,
    scratch_types=[
        pltpu.VMEM((EPT,), jnp.int32),
        pltpu.VMEM((EPT,), jnp.int32),
        pltpu.VMEM((EPT,), jnp.float32),
        pltpu.VMEM((CHB, D), jnp.float32),
        pltpu.VMEM((CHB, D), jnp.float32),
        pltpu.SemaphoreType.DMA,
        pltpu.SemaphoreType.DMA,
        pltpu.SemaphoreType.DMA,
        pltpu.SemaphoreType.DMA,
        pltpu.VMEM_SHARED((N_Y, D), jnp.float32),
    ],
)
def _sc_scatter(hm_hbm, src_hbm, dst_hbm, ex_hbm, agg_out,
                src_v, dst_v, ex_v, rows_a, rows_b, sem_a, sem_b,
                ssem_a, ssem_b, agg_sp):
    cid = lax.axis_index("c")
    sid = lax.axis_index("s")
    wid = sid * NC + cid
    ebase = wid * EPT

    pltpu.sync_copy(src_hbm.at[pl.ds(ebase, EPT)], src_v)
    pltpu.sync_copy(dst_hbm.at[pl.ds(ebase, EPT)], dst_v)
    pltpu.sync_copy(ex_hbm.at[pl.ds(ebase, EPT)], ex_v)

    def _zrow(r, _):
        for w in range(D // 16):
            rows_a[r, pl.ds(w * 16, 16)] = jnp.zeros((16,), jnp.float32)
        return 0
    lax.fori_loop(0, CHB, _zrow, 0)
    sbase = sid * NSPS

    def _zcopy(zi, _):
        pltpu.sync_copy(rows_a, agg_sp.at[pl.ds(sbase + zi * CHB, CHB)])
        return 0

    lax.fori_loop(0, NSPS // CHB, _zcopy, 0)
    pltpu.sync_copy(rows_a.at[pl.ds(0, NSPS % CHB)],
                    agg_sp.at[pl.ds(sbase + (NSPS // CHB) * CHB, NSPS % CHB)])
    plsc.subcore_barrier()

    def _gather(ci, rows, sem):
        return pltpu.async_copy(
            hm_hbm.at[src_v.at[pl.ds(ci * CHB, CHB)]], rows, sem)

    def _gwait(rows, sem):
        pltpu.make_async_copy(hm_hbm.at[pl.ds(0, CHB)], rows, sem).wait()

    def _scale(ci, rows):
        cbase = ci * CHB

        @plsc.parallel_loop(0, CHB, unroll=8)
        def _row(r):
            splat = plsc.load_gather(
                ex_v, [jnp.full((16,), cbase + r, jnp.int32)])
            for w in range(D // 16):
                sl = pl.ds(w * 16, 16)
                rows[r, sl] = rows[r, sl] * splat

    def _scat(ci, rows, sem):
        pltpu.async_copy(rows, agg_sp.at[dst_v.at[pl.ds(ci * CHB, CHB)]],
                         sem, add=True)

    def _swait(rows, sem):
        pltpu.make_async_copy(rows, agg_sp.at[pl.ds(0, CHB)], sem).wait()

    _gather(0, rows_a, sem_a)
    _gwait(rows_a, sem_a)
    _scale(0, rows_a)
    _gather(1, rows_b, sem_b)
    _scat(0, rows_a, ssem_a)

    def _pair(p, _):
        c1 = 2 * p + 1
        _gwait(rows_b, sem_b)
        _swait(rows_a, ssem_a)
        _gather(c1 + 1, rows_a, sem_a)
        _scale(c1, rows_b)
        _scat(c1, rows_b, ssem_b)
        _gwait(rows_a, sem_a)
        _swait(rows_b, ssem_b)

        @pl.when(p < (NCHB - 3) // 2)
        def _():
            _gather(c1 + 2, rows_b, sem_b)

        _scale(c1 + 1, rows_a)
        _scat(c1 + 1, rows_a, ssem_a)
        return 0

    lax.fori_loop(0, (NCHB - 1) // 2, _pair, 0)
    _swait(rows_a, ssem_a)

    plsc.subcore_barrier()
    pltpu.sync_copy(agg_sp.at[pl.ds(sbase, NSPS)],
                    agg_out.at[cid, pl.ds(sbase, NSPS)])


@functools.partial(
    pl.kernel,
    out_type=jax.ShapeDtypeStruct((E,), jnp.float32),
    mesh=_MESH,
    compiler_params=_SC_PARAMS,
    scratch_types=[
        pltpu.VMEM((NYP,), jnp.float32),
        pltpu.VMEM((EPT,), jnp.int32),
        pltpu.VMEM((EPT,), jnp.float32),
        pltpu.VMEM((EPT,), jnp.float32),
    ],
)
def _sc_alpha(ex_hbm, dst_hbm, den_hbm, a_out, den_v, dst_v, ex_v, a_v):
    ebase = _wid() * EPT
    pltpu.sync_copy(den_hbm, den_v)
    pltpu.sync_copy(dst_hbm.at[pl.ds(ebase, EPT)], dst_v)
    pltpu.sync_copy(ex_hbm.at[pl.ds(ebase, EPT)], ex_v)

    @plsc.parallel_loop(0, EPT // 16, unroll=4)
    def _group(g):
        i0 = g * 16
        d16 = plsc.load_gather(den_v, [dst_v[pl.ds(i0, 16)]])
        a_v[pl.ds(i0, 16)] = ex_v[pl.ds(i0, 16)] / (d16 + 1e-16)

    pltpu.sync_copy(a_v, a_out.at[pl.ds(ebase, EPT)])


@functools.partial(
    pl.kernel,
    out_type=jax.ShapeDtypeStruct((E * 7,), jnp.float32),
    mesh=_MESH,
    compiler_params=_SC_PARAMS,
    scratch_types=[
        pltpu.VMEM((EPT,), jnp.int32),
        pltpu.VMEM((EPT,), jnp.int32),
        pltpu.VMEM((16,), jnp.float32),
        pltpu.VMEM((CH, 16), jnp.float32),
        pltpu.VMEM((CH, 16), jnp.float32),
        pltpu.VMEM((CH * 7,), jnp.float32),
    ],
)
def _sc_rel(uv_hbm, src_hbm, dst_hbm, bg_hbm, rel_out, src_v, dst_v, bg_v,
            r1_v, r2_v, o_v):
    ebase = _wid() * EPT
    pltpu.sync_copy(src_hbm.at[pl.ds(ebase, EPT)], src_v)
    pltpu.sync_copy(dst_hbm.at[pl.ds(ebase, EPT)], dst_v)
    pltpu.sync_copy(bg_hbm, bg_v)
    bgv = bg_v[...]
    lane = jax.lax.iota(jnp.int32, 16)
    rhalf = jnp.where(lane >= 8, 1, 0)
    col1 = lane & 7
    col2 = col1 + 8
    cmask = col1 < 7

    def _chunk(ci, _):
        cbase = ci * CH
        pltpu.sync_copy(uv_hbm.at[src_v.at[pl.ds(cbase, CH)]], r1_v)
        pltpu.sync_copy(uv_hbm.at[dst_v.at[pl.ds(cbase, CH)]], r2_v)

        @plsc.parallel_loop(0, CH // 2, unroll=4)
        def _pair(r):
            row16 = jnp.full((16,), 2 * r, jnp.int32) + rhalf
            a = plsc.load_gather(r1_v, [row16, col1])
            b = plsc.load_gather(r2_v, [row16, col2])
            plsc.store_scatter(o_v, [row16 * 7 + col1], a + b + bgv,
                               mask=cmask)
        pltpu.sync_copy(o_v, rel_out.at[pl.ds((ebase + cbase) * 7, CH * 7)])
        return 0

    lax.fori_loop(0, EPT // CH, _chunk, 0)


def _tc_xmean_body(x_ref, xb_ref, out_ref, acc, cnt):
    i = pl.program_id(0)

    @pl.when(i == 0)
    def _():
        acc[...] = jnp.zeros_like(acc)
        cnt[...] = jnp.zeros_like(cnt)

    xb = xb_ref[0, 0, :]
    oh = (jax.lax.broadcasted_iota(jnp.int32, (B, xb.shape[0]), 0)
          == xb[None, :]).astype(jnp.float32)
    acc[...] += jax.lax.dot(oh, x_ref[...],
                            preferred_element_type=jnp.float32)
    cnt[...] += jnp.sum(oh, axis=1, keepdims=True)

    @pl.when(i == pl.num_programs(0) - 1)
    def _():
        out_ref[...] = acc[...] / jnp.maximum(
            jnp.broadcast_to(cnt[...], (B, D)), 1.0)


def _tc_xmean(x, xb3):
    nblk = N_X // 2000
    return pl.pallas_call(
        _tc_xmean_body,
        grid=(nblk,),
        in_specs=[pl.BlockSpec((2000, D), lambda i: (i, 0)),
                  pl.BlockSpec((1, 1, 2000), lambda i: (i, 0, 0))],
        out_specs=pl.BlockSpec((B, D), lambda i: (0, 0)),
        out_shape=jax.ShapeDtypeStruct((B, D), jnp.float32),
        scratch_shapes=[pltpu.VMEM((B, D), jnp.float32),
                        pltpu.VMEM((B, 1), jnp.float32)],
    )(x, xb3)


def _tc_hpass_body(y_ref, ws_ref, wm_ref, asrc_ref, adst_ref, ete_ref,
                   aet_ref, hs_ref, hm_ref, s1_ref, s2_ref, sh_ref, mx):
    i = pl.program_id(0)
    yb = y_ref[...]
    hs = jax.lax.dot(yb, ws_ref[...], preferred_element_type=jnp.float32)
    hm = jax.lax.dot(yb, wm_ref[...], preferred_element_type=jnp.float32)
    s1 = jax.lax.dot(hm, asrc_ref[...], preferred_element_type=jnp.float32)
    s2 = jax.lax.dot(hs, adst_ref[...], preferred_element_type=jnp.float32)
    hs_ref[...] = hs
    hm_ref[...] = hm
    s1_ref[...] = s1
    s2_ref[...] = s2

    @pl.when(i == 0)
    def _():
        mx[0] = -jnp.inf
        mx[1] = -jnp.inf

    mx[0] = jnp.maximum(mx[0], jnp.max(s1))
    mx[1] = jnp.maximum(mx[1], jnp.max(s2))

    @pl.when(i == pl.num_programs(0) - 1)
    def _():
        t4 = jax.lax.dot(ete_ref[...], aet_ref[...],
                         preferred_element_type=jnp.float32)
        sraw = mx[0] + mx[1] + jnp.max(t4)
        sh = jnp.where(sraw >= 0.0, sraw, sraw * 0.2)
        sh_ref[...] = jnp.concatenate(
            [t4, jnp.zeros((12, 1), jnp.float32),
             jnp.full((16, 1), sh, jnp.float32)], axis=0)


def _tc_hpass(y, ws, wm, asrc, adst, ete, aet):
    nblk = NYP // 512
    return pl.pallas_call(
        _tc_hpass_body,
        grid=(nblk,),
        in_specs=[pl.BlockSpec((512, D), lambda i: (i, 0)),
                  pl.BlockSpec((D, D), lambda i: (0, 0)),
                  pl.BlockSpec((D, D), lambda i: (0, 0)),
                  pl.BlockSpec((D, 1), lambda i: (0, 0)),
                  pl.BlockSpec((D, 1), lambda i: (0, 0)),
                  pl.BlockSpec((4, D), lambda i: (0, 0)),
                  pl.BlockSpec((D, 1), lambda i: (0, 0))],
        out_specs=[pl.BlockSpec((512, D), lambda i: (i, 0)),
                   pl.BlockSpec((512, D), lambda i: (i, 0)),
                   pl.BlockSpec((512, 1), lambda i: (i, 0)),
                   pl.BlockSpec((512, 1), lambda i: (i, 0)),
                   pl.BlockSpec((32, 1), lambda i: (0, 0))],
        out_shape=[jax.ShapeDtypeStruct((NYP, D), jnp.float32),
                   jax.ShapeDtypeStruct((NYP, D), jnp.float32),
                   jax.ShapeDtypeStruct((NYP, 1), jnp.float32),
                   jax.ShapeDtypeStruct((NYP, 1), jnp.float32),
                   jax.ShapeDtypeStruct((32, 1), jnp.float32)],
        scratch_shapes=[pltpu.SMEM((2,), jnp.float32)],
    )(y, ws, wm, asrc, adst, ete, aet)


FB = 256


def _tc_fin_body(hs_ref, agg_ref, d8_ref, xm_ref, wx_ref, ete_ref, yb_ref,
                 b_ref, wga_ref, wgb_ref, y_ref, yt_ref, den_ref, uva_ref,
                 uvb_ref):
    i = pl.program_id(0)
    aggc = agg_ref[0] + agg_ref[1]
    d8 = d8_ref[0] + d8_ref[1]
    den = d8[:, 0:1]
    beta = d8[:, 1:5]
    aggun = aggc + jax.lax.dot(beta, ete_ref[...],
                               preferred_element_type=jnp.float32)
    yb = yb_ref[0, 0, :]
    oh = (yb[:, None] == jax.lax.broadcasted_iota(
        jnp.int32, (yb.shape[0], B), 1)).astype(jnp.float32)
    xw = jax.lax.dot(xm_ref[...], wx_ref[...],
                     preferred_element_type=jnp.float32)
    ctx = jax.lax.dot(oh, xw, preferred_element_type=jnp.float32)
    ynew = hs_ref[...] + aggun / (den + 1e-16) + ctx + b_ref[...]
    ynew = jnp.maximum(ynew, 0.0)
    rid = i * FB + jax.lax.broadcasted_iota(jnp.int32, (FB, 1), 0)
    keep = rid < N_Y
    ynew = jnp.where(jnp.broadcast_to(keep, ynew.shape), ynew, 0.0)
    y_ref[...] = ynew
    yt_ref[...] = ynew
    den_ref[...] = jnp.where(keep, den, 1.0)
    uva_ref[...] = jax.lax.dot(ynew, wga_ref[...],
                               preferred_element_type=jnp.float32)
    uvb_ref[...] = jax.lax.dot(ynew, wgb_ref[...],
                               preferred_element_type=jnp.float32)


def _tc_finalize(hs, agg, d8, xm, wx, ete, yb3, b1, wga, wgb):
    nblk = NYP // FB
    return pl.pallas_call(
        _tc_fin_body,
        grid=(nblk,),
        in_specs=[pl.BlockSpec((FB, D), lambda i: (i, 0)),
                  pl.BlockSpec((NC, FB, D), lambda i: (0, i, 0)),
                  pl.BlockSpec((NC, FB, 8), lambda i: (0, i, 0)),
                  pl.BlockSpec((B, D), lambda i: (0, 0)),
                  pl.BlockSpec((D, D), lambda i: (0, 0)),
                  pl.BlockSpec((4, D), lambda i: (0, 0)),
                  pl.BlockSpec((1, 1, FB), lambda i: (i, 0, 0)),
                  pl.BlockSpec((1, D), lambda i: (0, 0)),
                  pl.BlockSpec((D, 16), lambda i: (0, 0)),
                  pl.BlockSpec((D, 16), lambda i: (0, 0))],
        out_specs=[pl.BlockSpec((FB, D), lambda i: (i, 0)),
                   pl.BlockSpec((FB, D), lambda i: (i, 0)),
                   pl.BlockSpec((FB, 1), lambda i: (i, 0)),
                   pl.BlockSpec((FB, 16), lambda i: (i, 0)),
                   pl.BlockSpec((FB, 16), lambda i: (i, 0))],
        out_shape=[jax.ShapeDtypeStruct((NYP, D), jnp.float32),
                   jax.ShapeDtypeStruct((N_Y, D), jnp.float32),
                   jax.ShapeDtypeStruct((NYP, 1), jnp.float32),
                   jax.ShapeDtypeStruct((NYP, 16), jnp.float32),
                   jax.ShapeDtypeStruct((NYP, 16), jnp.float32)],
    )(hs, agg, d8, xm, wx, ete, yb3, b1, wga, wgb)


def _tc_scores_body(y_ref, w_ref, b_ref, out_ref):
    out_ref[...] = (jax.lax.dot(y_ref[...], w_ref[...],
                                preferred_element_type=jnp.float32)
                    + b_ref[...])


def _tc_scores(y, wz, bz1):
    return pl.pallas_call(
        _tc_scores_body,
        grid=(20, 2),
        in_specs=[pl.BlockSpec((512, D), lambda i, j: (i, 0)),
                  pl.BlockSpec((D, 512), lambda i, j: (0, j)),
                  pl.BlockSpec((1, 512), lambda i, j: (0, j))],
        out_specs=pl.BlockSpec((512, 512), lambda i, j: (i, j)),
        out_shape=jax.ShapeDtypeStruct((N_Y, VOCAB), jnp.float32),
    )(y, wz, bz1)


def kernel(x, x_batch, tgt_y, tgt_edge_index, tgt_edge_type, tgt_y_batch,
           params):
    src = tgt_edge_index[0]
    dst = tgt_edge_index[1]
    et = tgt_edge_type

    tgt_pad = jnp.pad(tgt_y[:, 0], (0, NYP - N_Y))
    yb_pad = jnp.pad(tgt_y_batch, (0, NYP - N_Y)).reshape(NYP // FB, 1, FB)
    xb3 = x_batch.reshape(N_X // 2000, 1, 2000)

    y0 = _sc_gather_rows(params['emb'], tgt_pad)
    xm = _tc_xmean(x, xb3)

    wg = params['Wg']
    wgtop = jnp.pad(wg[:D], ((0, 0), (0, 1)))
    wgbot = jnp.pad(wg[D:], ((0, 0), (0, 1)))
    wga = jnp.concatenate([wgtop, wgbot], axis=1)
    wgb = jnp.concatenate([wgbot, wgtop], axis=1)
    bg16 = jnp.tile(jnp.pad(params['bg'], (0, 1)), 2)
    b_layers = [lp['b'].reshape(1, D) for lp in params['layers']]

    y = y0
    ytrim = None
    alphas = []
    uv = None
    for li in range(3):
        lp = params['layers'][li]
        hs, hm, s1, s2, sh32 = _tc_hpass(
            y, lp['W_self'], lp['W_msg'], lp['a_src'].reshape(D, 1),
            lp['a_dst'].reshape(D, 1), lp['et_emb'],
            lp['a_et'].reshape(D, 1))
        t16 = sh32[:16, 0]
        sh16 = sh32[16:, 0]
        ex, den8 = _sc_escore(s1.reshape(NYP), s2.reshape(NYP), t16, sh16,
                              src, dst, et)
        agg = _sc_scatter(hm, src, dst, ex)
        y, ytrim, den, uva, uvb = _tc_finalize(hs, agg, den8, xm, lp['W_x'],
                                               lp['et_emb'], yb_pad,
                                               b_layers[li], wga, wgb)
        if li < 2:
            alphas.append(_sc_alpha(ex, dst, den.reshape(NYP)))
        else:
            ex3, den3 = ex, den

    bz1 = params['bz'].reshape(1, VOCAB)
    embeds = _tc_scores(y0, params['Wz'], bz1)
    y_score = _tc_scores(y, params['Wz'], bz1)
    alphas.append(_sc_alpha(ex3, dst, den3.reshape(NYP)))
    relp = _sc_rel(uva, src, dst, bg16)
    rel = relp.reshape(E, 7)

    return (ytrim, tgt_edge_index, tgt_edge_type, y_score, rel, embeds,
            alphas[0], alphas[1], alphas[2])


# speedup vs baseline: 1.0047x; 1.0047x over previous
import functools

import jax
import jax.numpy as jnp
from jax import lax
from jax.experimental import pallas as pl
from jax.experimental.pallas import tpu as pltpu
from jax.experimental.pallas import tpu_sc as plsc

N_X = 100000
N_Y = 10000
NYP = 10240
E = 320000
D = 128
W = 144
VOCAB = 1000
VP = 1024
B = 64
NC, NS = 2, 16
NTILES = NC * NS
EPT = E // NTILES
CH = 400
NPT = NYP // NTILES
NPS = NYP // NS

_MESH = plsc.VectorSubcoreMesh(
    core_axis_name="c", subcore_axis_name="s", num_cores=NC, num_subcores=NS)
_SC_PARAMS = pltpu.CompilerParams(needs_layout_passes=False,
                                  use_tc_tiling_on_sc=False)


def _wid():
    return lax.axis_index("s") * NC + lax.axis_index("c")


@functools.partial(
    pl.kernel,
    out_type=jax.ShapeDtypeStruct((NYP, D), jnp.float32),
    mesh=_MESH,
    compiler_params=_SC_PARAMS,
    scratch_types=[pltpu.VMEM((NPT,), jnp.int32),
                   pltpu.VMEM((NPT, D), jnp.float32)],
)
def _sc_gather_rows(emb_hbm, idx_hbm, out_hbm, idx_v, rows_v):
    base = _wid() * NPT
    pltpu.sync_copy(idx_hbm.at[pl.ds(base, NPT)], idx_v)
    pltpu.sync_copy(emb_hbm.at[idx_v], rows_v)
    pltpu.sync_copy(rows_v, out_hbm.at[pl.ds(base, NPT)])


CHE = 2000


@functools.partial(
    pl.kernel,
    out_type=(jax.ShapeDtypeStruct((E,), jnp.float32),
              jax.ShapeDtypeStruct((NC, NYP, 8), jnp.float32)),
    mesh=_MESH,
    compiler_params=_SC_PARAMS,
    scratch_types=[
        pltpu.VMEM((NYP,), jnp.float32),
        pltpu.VMEM((NYP,), jnp.float32),
        pltpu.VMEM((16,), jnp.float32),
        pltpu.VMEM((16,), jnp.float32),
        pltpu.VMEM((CHE,), jnp.int32),
        pltpu.VMEM((CHE,), jnp.int32),
        pltpu.VMEM((CHE,), jnp.int32),
        pltpu.VMEM((CHE,), jnp.float32),
        pltpu.VMEM((CHE, 8), jnp.float32),
        pltpu.SemaphoreType.DMA,
        pltpu.SemaphoreType.DMA,
        pltpu.SemaphoreType.DMA,
        pltpu.VMEM_SHARED((N_Y, 8), jnp.float32),
    ],
)
def _sc_escore(s1_hbm, s2_hbm, t_hbm, sh_hbm, src_hbm, dst_hbm, et_hbm,
               ex_out, den8_out, s1_v, s2_v, t_v, sh_v, src_v, dst_v, et_v,
               ex_v, d8_v, isem1, isem2, isem3, d8_sp):
    cid = lax.axis_index("c")
    sid = lax.axis_index("s")
    wid = sid * NC + cid
    ebase = wid * EPT
    pltpu.sync_copy(s1_hbm, s1_v)
    pltpu.sync_copy(s2_hbm, s2_v)
    pltpu.sync_copy(t_hbm, t_v)
    pltpu.sync_copy(sh_hbm, sh_v)
    shv = sh_v[...]
    lane = jax.lax.iota(jnp.int32, 16)
    zidx = jnp.zeros((16,), jnp.int32)
    zval = jnp.zeros((16,), jnp.float32)
    zrow = lane // 8
    zcol = lane & 7

    def _zero2(g, _):
        plsc.store_scatter(d8_v, [zrow + g * 2, zcol], zval)
        return 0

    lax.fori_loop(0, CHE // 2, _zero2, 0)
    sbase = sid * NSPS
    pltpu.sync_copy(d8_v.at[pl.ds(0, NSPS)], d8_sp.at[pl.ds(sbase, NSPS)])
    plsc.subcore_barrier()

    def _chunk(ci, _):
        cbase = ci * CHE
        c1 = pltpu.async_copy(src_hbm.at[pl.ds(ebase + cbase, CHE)], src_v,
                              isem1)
        c2 = pltpu.async_copy(dst_hbm.at[pl.ds(ebase + cbase, CHE)], dst_v,
                              isem2)
        c3 = pltpu.async_copy(et_hbm.at[pl.ds(ebase + cbase, CHE)], et_v,
                              isem3)
        c1.wait()
        c2.wait()
        c3.wait()

        @plsc.parallel_loop(0, CHE // 16, unroll=4)
        def _group(g):
            i0 = g * 16
            i16 = lane + i0
            et16 = et_v[pl.ds(i0, 16)]
            e = (plsc.load_gather(s1_v, [src_v[pl.ds(i0, 16)]])
                 + plsc.load_gather(s2_v, [dst_v[pl.ds(i0, 16)]])
                 + plsc.load_gather(t_v, [et16]))
            e = jnp.where(e >= 0.0, e, e * 0.2)
            ex = jnp.exp(e - shv)
            ex_v[pl.ds(i0, 16)] = ex
            plsc.store_scatter(d8_v, [i16, zidx], ex)
            plsc.store_scatter(d8_v, [i16, et16 + 1], ex)

        pltpu.sync_copy(ex_v, ex_out.at[pl.ds(ebase + cbase, CHE)])
        pltpu.sync_copy(d8_v, d8_sp.at[dst_v], add=True)

        @plsc.parallel_loop(0, CHE // 16, unroll=4)
        def _unwrite(g):
            i0 = g * 16
            i16 = lane + i0
            et16 = et_v[pl.ds(i0, 16)]
            plsc.store_scatter(d8_v, [i16, zidx], zval)
            plsc.store_scatter(d8_v, [i16, et16 + 1], zval)

        return 0

    lax.fori_loop(0, EPT // CHE, _chunk, 0)
    plsc.subcore_barrier()
    pltpu.sync_copy(d8_sp.at[pl.ds(sbase, NSPS)],
                    den8_out.at[cid, pl.ds(sbase, NSPS)])


CHB = 80
NCHB = EPT // CHB
NSPS = N_Y // NS


@functools.partial(
    pl.kernel,
    out_type=jax.ShapeDtypeStruct((NC, NYP, D), jnp.float32),
    mesh=_MESH,
    compiler_params=_SC_PARAMS,
    scratch_types=[
        pltpu.VMEM((EPT,), jnp.int32),
        pltpu.VMEM((EPT,), jnp.int32),
        pltpu.VMEM((EPT,), jnp.float32),
        pltpu.VMEM((CHB, D), jnp.float32),
        pltpu.VMEM((CHB, D), jnp.float32),
        pltpu.SemaphoreType.DMA,
        pltpu.SemaphoreType.DMA,
        pltpu.SemaphoreType.DMA,
        pltpu.SemaphoreType.DMA,
        pltpu.VMEM_SHARED((N_Y, D), jnp.float32),
    ],
)
def _sc_scatter(hm_hbm, src_hbm, dst_hbm, ex_hbm, agg_out,
                src_v, dst_v, ex_v, rows_a, rows_b, sem_a, sem_b,
                ssem_a, ssem_b, agg_sp):
    cid = lax.axis_index("c")
    sid = lax.axis_index("s")
    wid = sid * NC + cid
    ebase = wid * EPT

    pltpu.sync_copy(src_hbm.at[pl.ds(ebase, EPT)], src_v)
    pltpu.sync_copy(dst_hbm.at[pl.ds(ebase, EPT)], dst_v)
    pltpu.sync_copy(ex_hbm.at[pl.ds(ebase, EPT)], ex_v)

    def _zrow(r, _):
        for w in range(D // 16):
            rows_a[r, pl.ds(w * 16, 16)] = jnp.zeros((16,), jnp.float32)
        return 0
    lax.fori_loop(0, CHB, _zrow, 0)
    sbase = sid * NSPS

    def _zcopy(zi, _):
        pltpu.sync_copy(rows_a, agg_sp.at[pl.ds(sbase + zi * CHB, CHB)])
        return 0

    lax.fori_loop(0, NSPS // CHB, _zcopy, 0)
    pltpu.sync_copy(rows_a.at[pl.ds(0, NSPS % CHB)],
                    agg_sp.at[pl.ds(sbase + (NSPS // CHB) * CHB, NSPS % CHB)])
    plsc.subcore_barrier()

    def _gather(ci, rows, sem):
        return pltpu.async_copy(
            hm_hbm.at[src_v.at[pl.ds(ci * CHB, CHB)]], rows, sem)

    def _gwait(rows, sem):
        pltpu.make_async_copy(hm_hbm.at[pl.ds(0, CHB)], rows, sem).wait()

    def _scale(ci, rows):
        cbase = ci * CHB

        @plsc.parallel_loop(0, CHB, unroll=4)
        def _row(r):
            splat = plsc.load_gather(
                ex_v, [jnp.full((16,), cbase + r, jnp.int32)])
            for w in range(D // 16):
                sl = pl.ds(w * 16, 16)
                rows[r, sl] = rows[r, sl] * splat

    def _scat(ci, rows, sem):
        pltpu.async_copy(rows, agg_sp.at[dst_v.at[pl.ds(ci * CHB, CHB)]],
                         sem, add=True)

    def _swait(rows, sem):
        pltpu.make_async_copy(rows, agg_sp.at[pl.ds(0, CHB)], sem).wait()

    _gather(0, rows_a, sem_a)
    _gwait(rows_a, sem_a)
    _scale(0, rows_a)
    _gather(1, rows_b, sem_b)
    _scat(0, rows_a, ssem_a)

    def _pair(p, _):
        c1 = 2 * p + 1
        _gwait(rows_b, sem_b)
        _swait(rows_a, ssem_a)
        _gather(c1 + 1, rows_a, sem_a)
        _scale(c1, rows_b)
        _scat(c1, rows_b, ssem_b)
        _gwait(rows_a, sem_a)
        _swait(rows_b, ssem_b)

        @pl.when(p < (NCHB - 3) // 2)
        def _():
            _gather(c1 + 2, rows_b, sem_b)

        _scale(c1 + 1, rows_a)
        _scat(c1 + 1, rows_a, ssem_a)
        return 0

    lax.fori_loop(0, (NCHB - 1) // 2, _pair, 0)
    _swait(rows_a, ssem_a)

    plsc.subcore_barrier()
    pltpu.sync_copy(agg_sp.at[pl.ds(sbase, NSPS)],
                    agg_out.at[cid, pl.ds(sbase, NSPS)])


@functools.partial(
    pl.kernel,
    out_type=jax.ShapeDtypeStruct((E,), jnp.float32),
    mesh=_MESH,
    compiler_params=_SC_PARAMS,
    scratch_types=[
        pltpu.VMEM((NYP,), jnp.float32),
        pltpu.VMEM((EPT,), jnp.int32),
        pltpu.VMEM((EPT,), jnp.float32),
        pltpu.VMEM((EPT,), jnp.float32),
    ],
)
def _sc_alpha(ex_hbm, dst_hbm, den_hbm, a_out, den_v, dst_v, ex_v, a_v):
    ebase = _wid() * EPT
    pltpu.sync_copy(den_hbm, den_v)
    pltpu.sync_copy(dst_hbm.at[pl.ds(ebase, EPT)], dst_v)
    pltpu.sync_copy(ex_hbm.at[pl.ds(ebase, EPT)], ex_v)

    @plsc.parallel_loop(0, EPT // 16, unroll=4)
    def _group(g):
        i0 = g * 16
        d16 = plsc.load_gather(den_v, [dst_v[pl.ds(i0, 16)]])
        a_v[pl.ds(i0, 16)] = ex_v[pl.ds(i0, 16)] / (d16 + 1e-16)

    pltpu.sync_copy(a_v, a_out.at[pl.ds(ebase, EPT)])


@functools.partial(
    pl.kernel,
    out_type=jax.ShapeDtypeStruct((E * 7,), jnp.float32),
    mesh=_MESH,
    compiler_params=_SC_PARAMS,
    scratch_types=[
        pltpu.VMEM((EPT,), jnp.int32),
        pltpu.VMEM((EPT,), jnp.int32),
        pltpu.VMEM((16,), jnp.float32),
        pltpu.VMEM((CH, 16), jnp.float32),
        pltpu.VMEM((CH, 16), jnp.float32),
        pltpu.VMEM((CH * 7,), jnp.float32),
    ],
)
def _sc_rel(uv_hbm, src_hbm, dst_hbm, bg_hbm, rel_out, src_v, dst_v, bg_v,
            r1_v, r2_v, o_v):
    ebase = _wid() * EPT
    pltpu.sync_copy(src_hbm.at[pl.ds(ebase, EPT)], src_v)
    pltpu.sync_copy(dst_hbm.at[pl.ds(ebase, EPT)], dst_v)
    pltpu.sync_copy(bg_hbm, bg_v)
    bgv = bg_v[...]
    lane = jax.lax.iota(jnp.int32, 16)
    rhalf = jnp.where(lane >= 8, 1, 0)
    col1 = lane & 7
    col2 = col1 + 8
    cmask = col1 < 7

    def _chunk(ci, _):
        cbase = ci * CH
        pltpu.sync_copy(uv_hbm.at[src_v.at[pl.ds(cbase, CH)]], r1_v)
        pltpu.sync_copy(uv_hbm.at[dst_v.at[pl.ds(cbase, CH)]], r2_v)

        @plsc.parallel_loop(0, CH // 2, unroll=4)
        def _pair(r):
            row16 = jnp.full((16,), 2 * r, jnp.int32) + rhalf
            a = plsc.load_gather(r1_v, [row16, col1])
            b = plsc.load_gather(r2_v, [row16, col2])
            plsc.store_scatter(o_v, [row16 * 7 + col1], a + b + bgv,
                               mask=cmask)
        pltpu.sync_copy(o_v, rel_out.at[pl.ds((ebase + cbase) * 7, CH * 7)])
        return 0

    lax.fori_loop(0, EPT // CH, _chunk, 0)


def _tc_xmean_body(x_ref, xb_ref, out_ref, acc, cnt):
    i = pl.program_id(0)

    @pl.when(i == 0)
    def _():
        acc[...] = jnp.zeros_like(acc)
        cnt[...] = jnp.zeros_like(cnt)

    xb = xb_ref[0, 0, :]
    oh = (jax.lax.broadcasted_iota(jnp.int32, (B, xb.shape[0]), 0)
          == xb[None, :]).astype(jnp.float32)
    acc[...] += jax.lax.dot(oh, x_ref[...],
                            preferred_element_type=jnp.float32)
    cnt[...] += jnp.sum(oh, axis=1, keepdims=True)

    @pl.when(i == pl.num_programs(0) - 1)
    def _():
        out_ref[...] = acc[...] / jnp.maximum(
            jnp.broadcast_to(cnt[...], (B, D)), 1.0)


def _tc_xmean(x, xb3):
    nblk = N_X // 2000
    return pl.pallas_call(
        _tc_xmean_body,
        grid=(nblk,),
        in_specs=[pl.BlockSpec((2000, D), lambda i: (i, 0)),
                  pl.BlockSpec((1, 1, 2000), lambda i: (i, 0, 0))],
        out_specs=pl.BlockSpec((B, D), lambda i: (0, 0)),
        out_shape=jax.ShapeDtypeStruct((B, D), jnp.float32),
        scratch_shapes=[pltpu.VMEM((B, D), jnp.float32),
                        pltpu.VMEM((B, 1), jnp.float32)],
    )(x, xb3)


def _tc_hpass_body(y_ref, ws_ref, wm_ref, asrc_ref, adst_ref, ete_ref,
                   aet_ref, hs_ref, hm_ref, s1_ref, s2_ref, sh_ref, mx):
    i = pl.program_id(0)
    yb = y_ref[...]
    hs = jax.lax.dot(yb, ws_ref[...], preferred_element_type=jnp.float32)
    hm = jax.lax.dot(yb, wm_ref[...], preferred_element_type=jnp.float32)
    s1 = jax.lax.dot(hm, asrc_ref[...], preferred_element_type=jnp.float32)
    s2 = jax.lax.dot(hs, adst_ref[...], preferred_element_type=jnp.float32)
    hs_ref[...] = hs
    hm_ref[...] = hm
    s1_ref[...] = s1
    s2_ref[...] = s2

    @pl.when(i == 0)
    def _():
        mx[0] = -jnp.inf
        mx[1] = -jnp.inf

    mx[0] = jnp.maximum(mx[0], jnp.max(s1))
    mx[1] = jnp.maximum(mx[1], jnp.max(s2))

    @pl.when(i == pl.num_programs(0) - 1)
    def _():
        t4 = jax.lax.dot(ete_ref[...], aet_ref[...],
                         preferred_element_type=jnp.float32)
        sraw = mx[0] + mx[1] + jnp.max(t4)
        sh = jnp.where(sraw >= 0.0, sraw, sraw * 0.2)
        sh_ref[...] = jnp.concatenate(
            [t4, jnp.zeros((12, 1), jnp.float32),
             jnp.full((16, 1), sh, jnp.float32)], axis=0)


def _tc_hpass(y, ws, wm, asrc, adst, ete, aet):
    nblk = NYP // 512
    return pl.pallas_call(
        _tc_hpass_body,
        grid=(nblk,),
        in_specs=[pl.BlockSpec((512, D), lambda i: (i, 0)),
                  pl.BlockSpec((D, D), lambda i: (0, 0)),
                  pl.BlockSpec((D, D), lambda i: (0, 0)),
                  pl.BlockSpec((D, 1), lambda i: (0, 0)),
                  pl.BlockSpec((D, 1), lambda i: (0, 0)),
                  pl.BlockSpec((4, D), lambda i: (0, 0)),
                  pl.BlockSpec((D, 1), lambda i: (0, 0))],
        out_specs=[pl.BlockSpec((512, D), lambda i: (i, 0)),
                   pl.BlockSpec((512, D), lambda i: (i, 0)),
                   pl.BlockSpec((512, 1), lambda i: (i, 0)),
                   pl.BlockSpec((512, 1), lambda i: (i, 0)),
                   pl.BlockSpec((32, 1), lambda i: (0, 0))],
        out_shape=[jax.ShapeDtypeStruct((NYP, D), jnp.float32),
                   jax.ShapeDtypeStruct((NYP, D), jnp.float32),
                   jax.ShapeDtypeStruct((NYP, 1), jnp.float32),
                   jax.ShapeDtypeStruct((NYP, 1), jnp.float32),
                   jax.ShapeDtypeStruct((32, 1), jnp.float32)],
        scratch_shapes=[pltpu.SMEM((2,), jnp.float32)],
    )(y, ws, wm, asrc, adst, ete, aet)


FB = 256


def _tc_fin_body(hs_ref, agg_ref, d8_ref, xm_ref, wx_ref, ete_ref, yb_ref,
                 b_ref, wga_ref, wgb_ref, y_ref, yt_ref, den_ref, uva_ref,
                 uvb_ref):
    i = pl.program_id(0)
    aggc = agg_ref[0] + agg_ref[1]
    d8 = d8_ref[0] + d8_ref[1]
    den = d8[:, 0:1]
    beta = d8[:, 1:5]
    aggun = aggc + jax.lax.dot(beta, ete_ref[...],
                               preferred_element_type=jnp.float32)
    yb = yb_ref[0, 0, :]
    oh = (yb[:, None] == jax.lax.broadcasted_iota(
        jnp.int32, (yb.shape[0], B), 1)).astype(jnp.float32)
    xw = jax.lax.dot(xm_ref[...], wx_ref[...],
                     preferred_element_type=jnp.float32)
    ctx = jax.lax.dot(oh, xw, preferred_element_type=jnp.float32)
    ynew = hs_ref[...] + aggun / (den + 1e-16) + ctx + b_ref[...]
    ynew = jnp.maximum(ynew, 0.0)
    rid = i * FB + jax.lax.broadcasted_iota(jnp.int32, (FB, 1), 0)
    keep = rid < N_Y
    ynew = jnp.where(jnp.broadcast_to(keep, ynew.shape), ynew, 0.0)
    y_ref[...] = ynew
    yt_ref[...] = ynew
    den_ref[...] = jnp.where(keep, den, 1.0)
    uva_ref[...] = jax.lax.dot(ynew, wga_ref[...],
                               preferred_element_type=jnp.float32)
    uvb_ref[...] = jax.lax.dot(ynew, wgb_ref[...],
                               preferred_element_type=jnp.float32)


def _tc_finalize(hs, agg, d8, xm, wx, ete, yb3, b1, wga, wgb):
    nblk = NYP // FB
    return pl.pallas_call(
        _tc_fin_body,
        grid=(nblk,),
        in_specs=[pl.BlockSpec((FB, D), lambda i: (i, 0)),
                  pl.BlockSpec((NC, FB, D), lambda i: (0, i, 0)),
                  pl.BlockSpec((NC, FB, 8), lambda i: (0, i, 0)),
                  pl.BlockSpec((B, D), lambda i: (0, 0)),
                  pl.BlockSpec((D, D), lambda i: (0, 0)),
                  pl.BlockSpec((4, D), lambda i: (0, 0)),
                  pl.BlockSpec((1, 1, FB), lambda i: (i, 0, 0)),
                  pl.BlockSpec((1, D), lambda i: (0, 0)),
                  pl.BlockSpec((D, 16), lambda i: (0, 0)),
                  pl.BlockSpec((D, 16), lambda i: (0, 0))],
        out_specs=[pl.BlockSpec((FB, D), lambda i: (i, 0)),
                   pl.BlockSpec((FB, D), lambda i: (i, 0)),
                   pl.BlockSpec((FB, 1), lambda i: (i, 0)),
                   pl.BlockSpec((FB, 16), lambda i: (i, 0)),
                   pl.BlockSpec((FB, 16), lambda i: (i, 0))],
        out_shape=[jax.ShapeDtypeStruct((NYP, D), jnp.float32),
                   jax.ShapeDtypeStruct((N_Y, D), jnp.float32),
                   jax.ShapeDtypeStruct((NYP, 1), jnp.float32),
                   jax.ShapeDtypeStruct((NYP, 16), jnp.float32),
                   jax.ShapeDtypeStruct((NYP, 16), jnp.float32)],
    )(hs, agg, d8, xm, wx, ete, yb3, b1, wga, wgb)


def _tc_scores_body(y_ref, w_ref, b_ref, out_ref):
    out_ref[...] = (jax.lax.dot(y_ref[...], w_ref[...],
                                preferred_element_type=jnp.float32)
                    + b_ref[...])


def _tc_scores(y, wz, bz1):
    return pl.pallas_call(
        _tc_scores_body,
        grid=(20, 2),
        in_specs=[pl.BlockSpec((512, D), lambda i, j: (i, 0)),
                  pl.BlockSpec((D, 512), lambda i, j: (0, j)),
                  pl.BlockSpec((1, 512), lambda i, j: (0, j))],
        out_specs=pl.BlockSpec((512, 512), lambda i, j: (i, j)),
        out_shape=jax.ShapeDtypeStruct((N_Y, VOCAB), jnp.float32),
    )(y, wz, bz1)


def kernel(x, x_batch, tgt_y, tgt_edge_index, tgt_edge_type, tgt_y_batch,
           params):
    src = tgt_edge_index[0]
    dst = tgt_edge_index[1]
    et = tgt_edge_type

    tgt_pad = jnp.pad(tgt_y[:, 0], (0, NYP - N_Y))
    yb_pad = jnp.pad(tgt_y_batch, (0, NYP - N_Y)).reshape(NYP // FB, 1, FB)
    xb3 = x_batch.reshape(N_X // 2000, 1, 2000)

    y0 = _sc_gather_rows(params['emb'], tgt_pad)
    xm = _tc_xmean(x, xb3)

    wg = params['Wg']
    wgtop = jnp.pad(wg[:D], ((0, 0), (0, 1)))
    wgbot = jnp.pad(wg[D:], ((0, 0), (0, 1)))
    wga = jnp.concatenate([wgtop, wgbot], axis=1)
    wgb = jnp.concatenate([wgbot, wgtop], axis=1)
    bg16 = jnp.tile(jnp.pad(params['bg'], (0, 1)), 2)
    b_layers = [lp['b'].reshape(1, D) for lp in params['layers']]

    y = y0
    ytrim = None
    alphas = []
    uv = None
    for li in range(3):
        lp = params['layers'][li]
        hs, hm, s1, s2, sh32 = _tc_hpass(
            y, lp['W_self'], lp['W_msg'], lp['a_src'].reshape(D, 1),
            lp['a_dst'].reshape(D, 1), lp['et_emb'],
            lp['a_et'].reshape(D, 1))
        t16 = sh32[:16, 0]
        sh16 = sh32[16:, 0]
        ex, den8 = _sc_escore(s1.reshape(NYP), s2.reshape(NYP), t16, sh16,
                              src, dst, et)
        agg = _sc_scatter(hm, src, dst, ex)
        y, ytrim, den, uva, uvb = _tc_finalize(hs, agg, den8, xm, lp['W_x'],
                                               lp['et_emb'], yb_pad,
                                               b_layers[li], wga, wgb)
        if li < 2:
            alphas.append(_sc_alpha(ex, dst, den.reshape(NYP)))
        else:
            ex3, den3 = ex, den

    bz1 = params['bz'].reshape(1, VOCAB)
    embeds = _tc_scores(y0, params['Wz'], bz1)
    y_score = _tc_scores(y, params['Wz'], bz1)
    alphas.append(_sc_alpha(ex3, dst, den3.reshape(NYP)))
    relp = _sc_rel(uva, src, dst, bg16)
    rel = relp.reshape(E, 7)

    return (ytrim, tgt_edge_index, tgt_edge_type, y_score, rel, embeds,
            alphas[0], alphas[1], alphas[2])
